# baseline (speedup 1.0000x reference)
import functools

import jax
import jax.numpy as jnp
from jax import lax
from jax.experimental import pallas as pl
from jax.experimental.pallas import tpu as pltpu
from jax.experimental.pallas import tpu_sc as plsc

EMB = 64
LANES = 16
NC, NS = 2, 16
NW = NC * NS
CB = 64

TR_BLK = 512
TR_GRID = 977
P_HALF = TR_BLK * TR_GRID


def _tr_body(x1_ref, x2_ref, o_ref):
    o_ref[:, 0:64] = x1_ref[...].T
    o_ref[:, 64:128] = x2_ref[...].T


def _repack_table(emb):
    emb_t = emb.T
    paired = pl.pallas_call(
        _tr_body,
        out_shape=jax.ShapeDtypeStruct((P_HALF, 128), jnp.float32),
        grid=(TR_GRID,),
        in_specs=[pl.BlockSpec((64, TR_BLK), lambda i: (0, i)),
                  pl.BlockSpec((64, TR_BLK), lambda i: (0, TR_GRID + i))],
        out_specs=pl.BlockSpec((TR_BLK, 128), lambda i: (i, 0)),
    )(emb_t, emb_t)
    return paired.reshape(2 * P_HALF, EMB)


def _remap_idx(w):
    return jnp.where(w < P_HALF, 2 * w, 2 * (w - P_HALF) + 1)


def _sc_scores_kernel(B, K, VROWS):
    KP1 = K + 1
    BPW = B // NW
    NCHUNK = BPW // CB
    mesh = plsc.VectorSubcoreMesh(core_axis_name="c", subcore_axis_name="s")

    @functools.partial(
        pl.kernel,
        out_type=jax.ShapeDtypeStruct((NW, NCHUNK, KP1 * CB), jnp.float32),
        mesh=mesh,
        scratch_types=[
            pltpu.VMEM((KP1, BPW), jnp.int32),
            pltpu.VMEM((BPW,), jnp.int32),
            pltpu.VMEM((CB, EMB), jnp.float32),
            pltpu.VMEM((KP1 * CB, EMB), jnp.float32),
            pltpu.VMEM((KP1 * CB,), jnp.float32),
            pltpu.SemaphoreType.DMA,
        ],
        compiler_params=pltpu.CompilerParams(
            needs_layout_passes=False, use_tc_tiling_on_sc=False),
    )
    def sc_kernel(uidx_hbm, cidx_hbm, emb_u, emb_v, out_hbm,
                  uidx_v, cidx_v, crows_v, urows_v, scores_v, sem):
        w = lax.axis_index("s") * NC + lax.axis_index("c")
        pltpu.sync_copy(uidx_hbm.at[w], uidx_v)
        pltpu.sync_copy(cidx_hbm.at[w], cidx_v)

        lanes = lax.iota(jnp.int32, LANES)

        def chunk_body(ch, _):
            base = ch * CB
            copies = [pltpu.async_copy(
                emb_v.at[cidx_v.at[pl.ds(base, CB)]], crows_v, sem)]
            for k in range(KP1):
                copies.append(pltpu.async_copy(
                    emb_u.at[uidx_v.at[k, pl.ds(base, CB)]],
                    urows_v.at[pl.ds(k * CB, CB)], sem))
            for c in copies:
                c.wait()

            for g in range(CB // LANES):
                blrow = g * LANES + lanes

                def d_body(d, accs):
                    dsp = jnp.full((LANES,), d, jnp.int32)
                    cd = plsc.load_gather(crows_v, [blrow, dsp])
                    return tuple(
                        accs[k] + plsc.load_gather(
                            urows_v, [blrow + (k * CB), dsp]) * cd
                        for k in range(KP1))

                accs = lax.fori_loop(
                    0, EMB, d_body,
                    tuple(jnp.zeros((LANES,), jnp.float32)
                          for _ in range(KP1)))
                for k in range(KP1):
                    scores_v[pl.ds(k * CB + g * LANES, LANES)] = (
                        accs[k] if k == 0 else -accs[k])

            pltpu.sync_copy(scores_v, out_hbm.at[w, ch])
            return 0

        lax.fori_loop(0, NCHUNK, chunk_body, 0)

    return sc_kernel


def _tc_loss_body(s_ref, o_ref):
    x = s_ref[...]
    ls = jnp.minimum(x, 0.0) - jnp.log(1.0 + jnp.exp(-jnp.abs(x)))
    o_ref[0, 0] = jnp.sum(ls)


def kernel(center_words, target_words, negative_words, embedding_u, embedding_v):
    B, K = negative_words.shape
    KP1 = K + 1
    BPW = B // NW

    lin_u = _repack_table(embedding_u)
    lin_v = _repack_table(embedding_v)

    u_idx = jnp.concatenate([target_words, negative_words], axis=1)
    u_idx = _remap_idx(u_idx).reshape(NW, BPW, KP1).transpose(0, 2, 1)
    c_idx = _remap_idx(center_words).reshape(NW, BPW)

    scores = _sc_scores_kernel(B, K, lin_u.shape[0])(u_idx, c_idx, lin_u, lin_v)
    total = B * KP1
    scores2d = scores.reshape(total // 128, 128)

    loss_sum = pl.pallas_call(
        _tc_loss_body,
        out_shape=jax.ShapeDtypeStruct((1, 1), jnp.float32),
        in_specs=[pl.BlockSpec(memory_space=pltpu.VMEM)],
        out_specs=pl.BlockSpec(memory_space=pltpu.SMEM),
    )(scores2d)
    return -loss_sum[0, 0] / B


# speedup vs baseline: 2.1238x; 2.1238x over previous
import functools

import jax
import jax.numpy as jnp
from jax import lax
from jax.experimental import pallas as pl
from jax.experimental.pallas import tpu as pltpu
from jax.experimental.pallas import tpu_sc as plsc

EMB = 64
LANES = 16
NC, NS = 2, 16
NW = NC * NS
CB = 64

VOCAB = 1000000
TR_BLK = 4096
NBT = -(-VOCAB // TR_BLK)
TR_GRID = -(-NBT // 2)
P_HALF = TR_BLK * TR_GRID


def _tr_body(x1_ref, x2_ref, o_ref):
    x = jnp.concatenate([x1_ref[...], x2_ref[...]], axis=0)
    o_ref[...] = x.T


def _repack_table(emb):
    emb_t = emb.T
    paired = pl.pallas_call(
        _tr_body,
        out_shape=jax.ShapeDtypeStruct((P_HALF, 128), jnp.float32),
        grid=(TR_GRID,),
        in_specs=[pl.BlockSpec((64, TR_BLK), lambda i: (0, i)),
                  pl.BlockSpec((64, TR_BLK), lambda i: (0, NBT - 1 - i))],
        out_specs=pl.BlockSpec((TR_BLK, 128), lambda i: (i, 0)),
    )(emb_t, emb_t)
    return paired.reshape(2 * P_HALF, EMB)


def _remap_idx(w):
    blk = w // TR_BLK
    rev = (NBT - 1 - blk) * TR_BLK + w % TR_BLK
    return jnp.where(blk < TR_GRID, 2 * w, 2 * rev + 1)


def _sc_scores_kernel(B, K, VROWS):
    KP1 = K + 1
    BPW = B // NW
    NCHUNK = BPW // CB
    mesh = plsc.VectorSubcoreMesh(core_axis_name="c", subcore_axis_name="s")

    @functools.partial(
        pl.kernel,
        out_type=jax.ShapeDtypeStruct((NW, NCHUNK, KP1 * CB), jnp.float32),
        mesh=mesh,
        scratch_types=[
            pltpu.VMEM((KP1, BPW), jnp.int32),
            pltpu.VMEM((BPW,), jnp.int32),
            pltpu.VMEM((CB, EMB), jnp.float32),
            pltpu.VMEM((KP1 * CB, EMB), jnp.float32),
            pltpu.VMEM((KP1 * CB,), jnp.float32),
            pltpu.SemaphoreType.DMA,
        ],
        compiler_params=pltpu.CompilerParams(
            needs_layout_passes=False, use_tc_tiling_on_sc=False),
    )
    def sc_kernel(uidx_hbm, cidx_hbm, emb_u, emb_v, out_hbm,
                  uidx_v, cidx_v, crows_v, urows_v, scores_v, sem):
        w = lax.axis_index("s") * NC + lax.axis_index("c")
        pltpu.sync_copy(uidx_hbm.at[w], uidx_v)
        pltpu.sync_copy(cidx_hbm.at[w], cidx_v)

        lanes = lax.iota(jnp.int32, LANES)

        def chunk_body(ch, _):
            base = ch * CB
            copies = [pltpu.async_copy(
                emb_v.at[cidx_v.at[pl.ds(base, CB)]], crows_v, sem)]
            for k in range(KP1):
                copies.append(pltpu.async_copy(
                    emb_u.at[uidx_v.at[k, pl.ds(base, CB)]],
                    urows_v.at[pl.ds(k * CB, CB)], sem))
            for c in copies:
                c.wait()

            for g in range(CB // LANES):
                blrow = g * LANES + lanes

                def d_body(d, accs):
                    dsp = jnp.full((LANES,), d, jnp.int32)
                    cd = plsc.load_gather(crows_v, [blrow, dsp])
                    return tuple(
                        accs[k] + plsc.load_gather(
                            urows_v, [blrow + (k * CB), dsp]) * cd
                        for k in range(KP1))

                accs = lax.fori_loop(
                    0, EMB, d_body,
                    tuple(jnp.zeros((LANES,), jnp.float32)
                          for _ in range(KP1)))
                for k in range(KP1):
                    scores_v[pl.ds(k * CB + g * LANES, LANES)] = (
                        accs[k] if k == 0 else -accs[k])

            pltpu.sync_copy(scores_v, out_hbm.at[w, ch])
            return 0

        lax.fori_loop(0, NCHUNK, chunk_body, 0)

    return sc_kernel


def _tc_loss_body(s_ref, o_ref):
    x = s_ref[...]
    ls = jnp.minimum(x, 0.0) - jnp.log(1.0 + jnp.exp(-jnp.abs(x)))
    o_ref[0, 0] = jnp.sum(ls)


def kernel(center_words, target_words, negative_words, embedding_u, embedding_v):
    B, K = negative_words.shape
    KP1 = K + 1
    BPW = B // NW

    lin_u = _repack_table(embedding_u)
    lin_v = _repack_table(embedding_v)

    u_idx = jnp.concatenate([target_words, negative_words], axis=1)
    u_idx = _remap_idx(u_idx).reshape(NW, BPW, KP1).transpose(0, 2, 1)
    c_idx = _remap_idx(center_words).reshape(NW, BPW)

    scores = _sc_scores_kernel(B, K, lin_u.shape[0])(u_idx, c_idx, lin_u, lin_v)
    total = B * KP1
    scores2d = scores.reshape(total // 128, 128)

    loss_sum = pl.pallas_call(
        _tc_loss_body,
        out_shape=jax.ShapeDtypeStruct((1, 1), jnp.float32),
        in_specs=[pl.BlockSpec(memory_space=pltpu.VMEM)],
        out_specs=pl.BlockSpec(memory_space=pltpu.SMEM),
    )(scores2d)
    return -loss_sum[0, 0] / B


# speedup vs baseline: 3.8868x; 1.8302x over previous
import functools

import jax
import jax.numpy as jnp
from jax import lax
from jax.experimental import pallas as pl
from jax.experimental.pallas import tpu as pltpu
from jax.experimental.pallas import tpu_sc as plsc

EMB = 64
LANES = 16
NC, NS = 2, 16
NW = NC * NS
CB = 32
NBUF = 2

VOCAB = 1000000
TR_BLK = 4096
NBT = -(-VOCAB // TR_BLK)
TR_GRID = -(-NBT // 2)
P_HALF = TR_BLK * TR_GRID


def _tr_body(x1_ref, x2_ref, o_ref):
    x = jnp.concatenate([x1_ref[...], x2_ref[...]], axis=0)
    o_ref[...] = x.T


def _repack_table(emb):
    emb_t = emb.T
    paired = pl.pallas_call(
        _tr_body,
        out_shape=jax.ShapeDtypeStruct((P_HALF, 128), jnp.float32),
        grid=(TR_GRID,),
        in_specs=[pl.BlockSpec((64, TR_BLK), lambda i: (0, i)),
                  pl.BlockSpec((64, TR_BLK), lambda i: (0, NBT - 1 - i))],
        out_specs=pl.BlockSpec((TR_BLK, 128), lambda i: (i, 0)),
    )(emb_t, emb_t)
    return paired.reshape(2 * P_HALF, EMB)


def _remap_idx(w):
    blk = w // TR_BLK
    rev = (NBT - 1 - blk) * TR_BLK + w % TR_BLK
    return jnp.where(blk < TR_GRID, 2 * w, 2 * rev + 1)


def _sc_scores_kernel(B, K, VROWS):
    KP1 = K + 1
    BPW = B // NW
    NCHUNK = BPW // CB
    mesh = plsc.VectorSubcoreMesh(core_axis_name="c", subcore_axis_name="s")

    @functools.partial(
        pl.kernel,
        out_type=jax.ShapeDtypeStruct((NW, NCHUNK, KP1 * CB), jnp.float32),
        mesh=mesh,
        scratch_types=[
            pltpu.VMEM((KP1, BPW), jnp.int32),
            pltpu.VMEM((BPW,), jnp.int32),
            pltpu.VMEM((NBUF, CB, EMB), jnp.float32),
            pltpu.VMEM((NBUF, KP1 * CB, EMB), jnp.float32),
            pltpu.VMEM((NBUF, KP1 * CB), jnp.float32),
            pltpu.SemaphoreType.DMA,
            pltpu.SemaphoreType.DMA,
        ],
        compiler_params=pltpu.CompilerParams(
            needs_layout_passes=False, use_tc_tiling_on_sc=False),
    )
    def sc_kernel(uidx_hbm, cidx_hbm, emb_u, emb_v, out_hbm,
                  uidx_v, cidx_v, crows_v, urows_v, scores_v, *sems):
        w = lax.axis_index("s") * NC + lax.axis_index("c")
        pltpu.sync_copy(uidx_hbm.at[w], uidx_v)
        pltpu.sync_copy(cidx_hbm.at[w], cidx_v)

        lanes = lax.iota(jnp.int32, LANES)

        def issue(ch):
            buf = ch % NBUF
            base = ch * CB
            copies = [pltpu.async_copy(
                emb_v.at[cidx_v.at[pl.ds(base, CB)]], crows_v.at[buf],
                sems[buf])]
            for k in range(KP1):
                copies.append(pltpu.async_copy(
                    emb_u.at[uidx_v.at[k, pl.ds(base, CB)]],
                    urows_v.at[buf, pl.ds(k * CB, CB)], sems[buf]))
            return copies

        def compute(ch):
            buf = ch % NBUF
            crows = crows_v.at[buf]
            urows = urows_v.at[buf]
            for g in range(CB // LANES):
                blrow = g * LANES + lanes

                def d_body(d, accs):
                    dskew = (jnp.full((LANES,), d, jnp.int32) + lanes) & 63
                    cd = plsc.load_gather(crows, [blrow, dskew])
                    return tuple(
                        accs[k] + plsc.load_gather(
                            urows, [blrow + (k * CB), dskew]) * cd
                        for k in range(KP1))

                accs = lax.fori_loop(
                    0, EMB, d_body,
                    tuple(jnp.zeros((LANES,), jnp.float32)
                          for _ in range(KP1)))
                for k in range(KP1):
                    scores_v[buf, pl.ds(k * CB + g * LANES, LANES)] = (
                        accs[k] if k == 0 else -accs[k])

            pltpu.sync_copy(scores_v.at[buf], out_hbm.at[w, ch])

        pending = {0: issue(0)}
        for ch in range(NCHUNK):
            for c in pending.pop(ch):
                c.wait()
            if ch + 1 < NCHUNK:
                pending[ch + 1] = issue(ch + 1)
            compute(ch)

    return sc_kernel


def _tc_loss_body(s_ref, o_ref):
    x = s_ref[...]
    ls = jnp.minimum(x, 0.0) - jnp.log(1.0 + jnp.exp(-jnp.abs(x)))
    o_ref[0, 0] = jnp.sum(ls)


def kernel(center_words, target_words, negative_words, embedding_u, embedding_v):
    B, K = negative_words.shape
    KP1 = K + 1
    BPW = B // NW

    lin_u = _repack_table(embedding_u)
    lin_v = _repack_table(embedding_v)

    u_idx = jnp.concatenate([target_words, negative_words], axis=1)
    u_idx = _remap_idx(u_idx).reshape(NW, BPW, KP1).transpose(0, 2, 1)
    c_idx = _remap_idx(center_words).reshape(NW, BPW)

    scores = _sc_scores_kernel(B, K, lin_u.shape[0])(u_idx, c_idx, lin_u, lin_v)
    total = B * KP1
    scores2d = scores.reshape(total // 128, 128)

    loss_sum = pl.pallas_call(
        _tc_loss_body,
        out_shape=jax.ShapeDtypeStruct((1, 1), jnp.float32),
        in_specs=[pl.BlockSpec(memory_space=pltpu.VMEM)],
        out_specs=pl.BlockSpec(memory_space=pltpu.SMEM),
    )(scores2d)
    return -loss_sum[0, 0] / B


# speedup vs baseline: 4.4740x; 1.1511x over previous
import functools

import jax
import jax.numpy as jnp
from jax import lax
from jax.experimental import pallas as pl
from jax.experimental.pallas import tpu as pltpu
from jax.experimental.pallas import tpu_sc as plsc

EMB = 64
LANES = 16
NC, NS = 2, 16
NW = NC * NS
CB = 32
NBUF = 2

VOCAB = 1000000
TR_BLK = 4096
NBT = -(-VOCAB // TR_BLK)
TR_GRID = -(-NBT // 2)
P_HALF = TR_BLK * TR_GRID


def _tr_body(xu1_ref, xu2_ref, xv1_ref, xv2_ref, ou_ref, ov_ref):
    xu = jnp.concatenate([xu1_ref[...], xu2_ref[...]], axis=0)
    ou_ref[...] = xu.T
    xv = jnp.concatenate([xv1_ref[...], xv2_ref[...]], axis=0)
    ov_ref[...] = xv.T


def _repack_tables(emb_u, emb_v):
    u_t, v_t = emb_u.T, emb_v.T
    fwd = pl.BlockSpec((64, TR_BLK), lambda i: (0, i))
    rev = pl.BlockSpec((64, TR_BLK), lambda i: (0, NBT - 1 - i))
    out = jax.ShapeDtypeStruct((P_HALF, 128), jnp.float32)
    pu, pv = pl.pallas_call(
        _tr_body,
        out_shape=(out, out),
        grid=(TR_GRID,),
        in_specs=[fwd, rev, fwd, rev],
        out_specs=(pl.BlockSpec((TR_BLK, 128), lambda i: (i, 0)),) * 2,
    )(u_t, u_t, v_t, v_t)
    return pu.reshape(2 * P_HALF, EMB), pv.reshape(2 * P_HALF, EMB)


def _remap_idx(w):
    blk = w // TR_BLK
    rev = (NBT - 1 - blk) * TR_BLK + w % TR_BLK
    return jnp.where(blk < TR_GRID, 2 * w, 2 * rev + 1)


def _sc_scores_kernel(B, K, VROWS):
    KP1 = K + 1
    BPW = B // NW
    NCHUNK = BPW // CB
    mesh = plsc.VectorSubcoreMesh(core_axis_name="c", subcore_axis_name="s")

    @functools.partial(
        pl.kernel,
        out_type=jax.ShapeDtypeStruct((NW, NCHUNK, KP1 * CB), jnp.float32),
        mesh=mesh,
        scratch_types=[
            pltpu.VMEM((KP1, BPW), jnp.int32),
            pltpu.VMEM((BPW,), jnp.int32),
            pltpu.VMEM((NBUF, CB, EMB), jnp.float32),
            pltpu.VMEM((NBUF, KP1 * CB, EMB), jnp.float32),
            pltpu.VMEM((NBUF, KP1 * CB), jnp.float32),
            pltpu.SemaphoreType.DMA,
            pltpu.SemaphoreType.DMA,
        ],
        compiler_params=pltpu.CompilerParams(
            needs_layout_passes=False, use_tc_tiling_on_sc=False),
    )
    def sc_kernel(uidx_hbm, cidx_hbm, emb_u, emb_v, out_hbm,
                  uidx_v, cidx_v, crows_v, urows_v, scores_v, *sems):
        w = lax.axis_index("s") * NC + lax.axis_index("c")
        pltpu.sync_copy(uidx_hbm.at[w], uidx_v)
        pltpu.sync_copy(cidx_hbm.at[w], cidx_v)

        lanes = lax.iota(jnp.int32, LANES)

        def issue(ch):
            buf = ch % NBUF
            base = ch * CB
            copies = [pltpu.async_copy(
                emb_v.at[cidx_v.at[pl.ds(base, CB)]], crows_v.at[buf],
                sems[buf])]
            for k in range(KP1):
                copies.append(pltpu.async_copy(
                    emb_u.at[uidx_v.at[k, pl.ds(base, CB)]],
                    urows_v.at[buf, pl.ds(k * CB, CB)], sems[buf]))
            return copies

        def compute(ch):
            buf = ch % NBUF
            crows = crows_v.at[buf]
            urows = urows_v.at[buf]
            for g in range(CB // LANES):
                blrow = g * LANES + lanes

                def d_body(d, accs):
                    dskew = (jnp.full((LANES,), d, jnp.int32) + lanes) & 63
                    cd = plsc.load_gather(crows, [blrow, dskew])
                    return tuple(
                        accs[k] + plsc.load_gather(
                            urows, [blrow + (k * CB), dskew]) * cd
                        for k in range(KP1))

                accs = lax.fori_loop(
                    0, EMB, d_body,
                    tuple(jnp.zeros((LANES,), jnp.float32)
                          for _ in range(KP1)))
                for k in range(KP1):
                    scores_v[buf, pl.ds(k * CB + g * LANES, LANES)] = (
                        accs[k] if k == 0 else -accs[k])

            pltpu.sync_copy(scores_v.at[buf], out_hbm.at[w, ch])

        pending = {0: issue(0)}
        for ch in range(NCHUNK):
            for c in pending.pop(ch):
                c.wait()
            if ch + 1 < NCHUNK:
                pending[ch + 1] = issue(ch + 1)
            compute(ch)

    return sc_kernel


def _tc_loss_body(s_ref, o_ref):
    x = s_ref[...]
    ls = jnp.minimum(x, 0.0) - jnp.log(1.0 + jnp.exp(-jnp.abs(x)))
    o_ref[0, 0] = jnp.sum(ls)


def kernel(center_words, target_words, negative_words, embedding_u, embedding_v):
    B, K = negative_words.shape
    KP1 = K + 1
    BPW = B // NW

    lin_u, lin_v = _repack_tables(embedding_u, embedding_v)

    u_idx = jnp.concatenate([target_words, negative_words], axis=1)
    u_idx = _remap_idx(u_idx).reshape(NW, BPW, KP1).transpose(0, 2, 1)
    c_idx = _remap_idx(center_words).reshape(NW, BPW)

    scores = _sc_scores_kernel(B, K, lin_u.shape[0])(u_idx, c_idx, lin_u, lin_v)
    total = B * KP1
    scores2d = scores.reshape(total // 128, 128)

    loss_sum = pl.pallas_call(
        _tc_loss_body,
        out_shape=jax.ShapeDtypeStruct((1, 1), jnp.float32),
        in_specs=[pl.BlockSpec(memory_space=pltpu.VMEM)],
        out_specs=pl.BlockSpec(memory_space=pltpu.SMEM),
    )(scores2d)
    return -loss_sum[0, 0] / B
